# baseline (speedup 1.0000x reference)
import functools

import jax
import jax.numpy as jnp
from jax import lax
from jax.experimental import pallas as pl
from jax.experimental.pallas import tpu as pltpu
from jax.experimental.pallas import tpu_sc as plsc


def _sc_gather(weight, ids):
    info = plsc.get_sparse_core_info()
    nc, ns = info.num_cores, info.num_subcores
    nw = nc * ns
    n = ids.shape[0]
    d = weight.shape[1]
    b_per_w = n // nw
    mesh = plsc.VectorSubcoreMesh(core_axis_name="c", subcore_axis_name="s")

    @functools.partial(
        pl.kernel,
        mesh=mesh,
        out_type=jax.ShapeDtypeStruct((n, d), jnp.float32),
        scratch_types=[
            pltpu.VMEM((b_per_w,), jnp.int32),
            pltpu.VMEM((b_per_w, d), jnp.float32),
            pltpu.SemaphoreType.DMA,
        ],
    )
    def gather_kernel(table_hbm, idx_hbm, out_hbm, idx_v, rows_v, sem):
        wid = lax.axis_index("s") * nc + lax.axis_index("c")
        base = wid * b_per_w
        pltpu.sync_copy(idx_hbm.at[pl.ds(base, b_per_w)], idx_v)
        pltpu.async_copy(table_hbm.at[idx_v], rows_v, sem).wait()
        pltpu.sync_copy(rows_v, out_hbm.at[pl.ds(base, b_per_w)])

    return gather_kernel(weight, ids)


def _matmul_body(x_ref, w_ref, o_ref):
    o_ref[...] = lax.dot_general(
        x_ref[...].astype(jnp.bfloat16), w_ref[...].astype(jnp.bfloat16),
        dimension_numbers=(((1,), (1,)), ((), ())),
        preferred_element_type=jnp.float32,
    )


def _tc_logits(x, weight, tv):
    n, e = x.shape
    v = weight.shape[0]
    grid = pl.cdiv(v, tv)
    return pl.pallas_call(
        _matmul_body,
        grid=(grid,),
        in_specs=[
            pl.BlockSpec((n, e), lambda i: (0, 0)),
            pl.BlockSpec((tv, e), lambda i: (i, 0)),
        ],
        out_specs=pl.BlockSpec((n, tv), lambda i: (0, i)),
        out_shape=jax.ShapeDtypeStruct((n, v), jnp.float32),
    )(x, weight)


def kernel(input_ids, weight):
    b, s = input_ids.shape
    v, e = weight.shape
    n = b * s
    ids = input_ids.reshape(n)
    x = _sc_gather(weight, ids)
    logits = _tc_logits(x, weight, tv=6144)
    return logits.reshape(b, s, v)


# speedup vs baseline: 1.0876x; 1.0876x over previous
import jax
import jax.numpy as jnp
from jax import lax
from jax.experimental import pallas as pl
from jax.experimental.pallas import tpu as pltpu


def _fused_body(ids_ref, w_any, w_tile, o_ref, x_vmem, sem):
    step = pl.program_id(0)

    @pl.when(step == 0)
    def _gather():
        n = x_vmem.shape[0]

        def issue(i, c):
            pltpu.make_async_copy(
                w_any.at[pl.ds(ids_ref[i], 1), :],
                x_vmem.at[pl.ds(i, 1), :],
                sem,
            ).start()
            return c

        lax.fori_loop(0, n, issue, 0, unroll=8)

        def drain(i, c):
            pltpu.make_async_copy(
                w_any.at[pl.ds(0, 1), :],
                x_vmem.at[pl.ds(0, 1), :],
                sem,
            ).wait()
            return c

        lax.fori_loop(0, n, drain, 0, unroll=8)

    o_ref[...] = lax.dot_general(
        x_vmem[...].astype(jnp.bfloat16), w_tile[...].astype(jnp.bfloat16),
        dimension_numbers=(((1,), (1,)), ((), ())),
        preferred_element_type=jnp.float32,
    )


def kernel(input_ids, weight):
    b, s = input_ids.shape
    v, e = weight.shape
    n = b * s
    tv = 6144
    ids = input_ids.reshape(n)
    grid_spec = pltpu.PrefetchScalarGridSpec(
        num_scalar_prefetch=1,
        grid=(pl.cdiv(v, tv),),
        in_specs=[
            pl.BlockSpec(memory_space=pl.ANY),
            pl.BlockSpec((tv, e), lambda i, ids_ref: (i, 0)),
        ],
        out_specs=pl.BlockSpec((n, tv), lambda i, ids_ref: (0, i)),
        scratch_shapes=[
            pltpu.VMEM((n, e), jnp.float32),
            pltpu.SemaphoreType.DMA,
        ],
    )
    logits = pl.pallas_call(
        _fused_body,
        grid_spec=grid_spec,
        out_shape=jax.ShapeDtypeStruct((n, v), jnp.float32),
    )(ids, weight, weight)
    return logits.reshape(b, s, v)


# speedup vs baseline: 1.1021x; 1.0133x over previous
import jax
import jax.numpy as jnp
from jax import lax
from jax.experimental import pallas as pl
from jax.experimental.pallas import tpu as pltpu


def _fused_body(ids_ref, w_any, w_tile, o_ref, x_vmem, sem):
    step = pl.program_id(0)

    @pl.when(step == 0)
    def _gather():
        n = x_vmem.shape[0]
        br, bc = ids_ref.shape

        def issue_row(r, c0):
            def issue(c, base):
                pltpu.make_async_copy(
                    w_any.at[pl.ds(ids_ref[r, c], 1), :],
                    x_vmem.at[pl.ds(base + c, 1), :],
                    sem,
                ).start()
                return base

            lax.fori_loop(0, bc, issue, c0, unroll=8)
            return c0 + bc

        lax.fori_loop(0, br, issue_row, 0)

        pltpu.make_async_copy(
            w_any.at[pl.ds(0, n), :], x_vmem, sem
        ).wait()

    o_ref[...] = lax.dot_general(
        x_vmem[...].astype(jnp.bfloat16), w_tile[...].astype(jnp.bfloat16),
        dimension_numbers=(((1,), (1,)), ((), ())),
        preferred_element_type=jnp.float32,
    )


def kernel(input_ids, weight):
    b, s = input_ids.shape
    v, e = weight.shape
    n = b * s
    tv = 6144
    grid_spec = pltpu.PrefetchScalarGridSpec(
        num_scalar_prefetch=1,
        grid=(pl.cdiv(v, tv),),
        in_specs=[
            pl.BlockSpec(memory_space=pl.ANY),
            pl.BlockSpec((tv, e), lambda i, ids_ref: (i, 0)),
        ],
        out_specs=pl.BlockSpec((n, tv), lambda i, ids_ref: (0, i)),
        scratch_shapes=[
            pltpu.VMEM((n, e), jnp.float32),
            pltpu.SemaphoreType.DMA,
        ],
    )
    logits = pl.pallas_call(
        _fused_body,
        grid_spec=grid_spec,
        out_shape=jax.ShapeDtypeStruct((n, v), jnp.float32),
    )(input_ids, weight, weight)
    return logits.reshape(b, s, v)


# speedup vs baseline: 1.1039x; 1.0016x over previous
import jax
import jax.numpy as jnp
from jax import lax
from jax.experimental import pallas as pl
from jax.experimental.pallas import tpu as pltpu


def _fused_body(ids_ref, w_any, w_tile, o_ref, x_vmem, sem):
    step = pl.program_id(0)

    @pl.when(step == 0)
    def _gather():
        n = x_vmem.shape[0]
        br, bc = ids_ref.shape

        def issue_row(r, c0):
            def issue(c, base):
                pltpu.make_async_copy(
                    w_any.at[pl.ds(ids_ref[r, c], 1), :],
                    x_vmem.at[pl.ds(base + c, 1), :],
                    sem,
                ).start()
                return base

            lax.fori_loop(0, bc, issue, c0, unroll=32)
            return c0 + bc

        lax.fori_loop(0, br, issue_row, 0)

        pltpu.make_async_copy(
            w_any.at[pl.ds(0, n), :], x_vmem, sem
        ).wait()

    o_ref[...] = lax.dot_general(
        x_vmem[...].astype(jnp.bfloat16), w_tile[...].astype(jnp.bfloat16),
        dimension_numbers=(((1,), (1,)), ((), ())),
        preferred_element_type=jnp.float32,
    )


def kernel(input_ids, weight):
    b, s = input_ids.shape
    v, e = weight.shape
    n = b * s
    tv = 6144
    grid_spec = pltpu.PrefetchScalarGridSpec(
        num_scalar_prefetch=1,
        grid=(pl.cdiv(v, tv),),
        in_specs=[
            pl.BlockSpec(memory_space=pl.ANY),
            pl.BlockSpec((tv, e), lambda i, ids_ref: (i, 0)),
        ],
        out_specs=pl.BlockSpec((n, tv), lambda i, ids_ref: (0, i)),
        scratch_shapes=[
            pltpu.VMEM((n, e), jnp.float32),
            pltpu.SemaphoreType.DMA,
        ],
    )
    logits = pl.pallas_call(
        _fused_body,
        grid_spec=grid_spec,
        out_shape=jax.ShapeDtypeStruct((n, v), jnp.float32),
    )(input_ids, weight, weight)
    return logits.reshape(b, s, v)
